# baseline (speedup 1.0000x reference)
import functools

import jax
import jax.numpy as jnp
from jax.experimental import pallas as pl
from jax.experimental.pallas import tpu as pltpu


def _round_up(x, m):
    return ((x + m - 1) // m) * m


def _conv_kernel(x_ref, w_ref, o_ref, xb_ref, rhs_ref, *,
                 offsets, cin, m, kpad, hw, ho, w, wo, bb):
    kc = cin * len(offsets)
    for b in range(bb):
        xb_ref[b] = x_ref[b].astype(jnp.bfloat16)
        for t, d in enumerate(offsets):
            md = min(m, hw - d)
            rhs_ref[t * cin:(t + 1) * cin, b * m:b * m + md] = (
                xb_ref[b, :, d:d + md])
    rhs_ref[kc:kpad, :] = jnp.ones((kpad - kc, bb * m), jnp.bfloat16)
    acc = jax.lax.dot_general(
        w_ref[...], rhs_ref[...],
        dimension_numbers=(((1,), (0,)), ((), ())),
        preferred_element_type=jnp.float32)
    for b in range(bb):
        for h in range(ho):
            o_ref[b, :, h * wo:(h + 1) * wo] = (
                acc[:, b * m + h * w:b * m + h * w + wo])


def kernel(x, weight, bias):
    N, Cin, H, W = x.shape
    Cout, Cin2, Kh, Kw = weight.shape
    assert Cin == Cin2
    Ho, Wo = H - Kh + 1, W - Kw + 1
    M = Ho * W
    offsets = tuple(kh * W + kw for kh in range(Kh) for kw in range(Kw))
    Kc = Cin * Kh * Kw
    Kpad = _round_up(Kc + 1, 8)
    B = 2 if N % 2 == 0 else 1

    w_mat = weight.transpose(0, 2, 3, 1).reshape(Cout, Kc)
    w_b = jnp.zeros((Cout, Kpad), jnp.bfloat16)
    w_b = w_b.at[:, :Kc].set(w_mat.astype(jnp.bfloat16))
    w_b = w_b.at[:, Kc].set(bias.astype(jnp.bfloat16))

    out = pl.pallas_call(
        functools.partial(_conv_kernel, offsets=offsets, cin=Cin, m=M,
                          kpad=Kpad, hw=H * W, ho=Ho, w=W, wo=Wo, bb=B),
        out_shape=jax.ShapeDtypeStruct((N, Cout, Ho * Wo), jnp.float32),
        grid=(N // B,),
        in_specs=[
            pl.BlockSpec((B, Cin, H * W), lambda n: (n, 0, 0)),
            pl.BlockSpec((Cout, Kpad), lambda n: (0, 0)),
        ],
        out_specs=pl.BlockSpec((B, Cout, Ho * Wo), lambda n: (n, 0, 0)),
        scratch_shapes=[
            pltpu.VMEM((B, Cin, H * W), jnp.bfloat16),
            pltpu.VMEM((Kpad, B * M), jnp.bfloat16),
        ],
        compiler_params=pltpu.CompilerParams(
            dimension_semantics=("parallel",),
        ),
    )(x.reshape(N, Cin, H * W), w_b)

    return out.reshape(N, Cout, Ho, Wo)


# speedup vs baseline: 1.0152x; 1.0152x over previous
import functools

import jax
import jax.numpy as jnp
from jax.experimental import pallas as pl
from jax.experimental.pallas import tpu as pltpu


def _round_up(x, m):
    return ((x + m - 1) // m) * m


def _conv_kernel(x_ref, w_ref, o_ref, xb_ref, rhs_ref, *,
                 offsets, cin, m, kpad, hw, ho, w, wo, bb):
    kc = cin * len(offsets)
    for b in range(bb):
        xb_ref[b] = x_ref[b].astype(jnp.bfloat16)
        for t, d in enumerate(offsets):
            md = min(m, hw - d)
            rhs_ref[t * cin:(t + 1) * cin, b * m:b * m + md] = (
                xb_ref[b, :, d:d + md])
    rhs_ref[kc:kpad, :] = jnp.ones((kpad - kc, bb * m), jnp.bfloat16)
    acc = jax.lax.dot_general(
        w_ref[...], rhs_ref[...],
        dimension_numbers=(((1,), (0,)), ((), ())),
        preferred_element_type=jnp.float32)
    for b in range(bb):
        for h in range(ho):
            o_ref[b, :, h * wo:(h + 1) * wo] = (
                acc[:, b * m + h * w:b * m + h * w + wo])


def kernel(x, weight, bias):
    N, Cin, H, W = x.shape
    Cout, Cin2, Kh, Kw = weight.shape
    assert Cin == Cin2
    Ho, Wo = H - Kh + 1, W - Kw + 1
    M = Ho * W
    offsets = tuple(kh * W + kw for kh in range(Kh) for kw in range(Kw))
    Kc = Cin * Kh * Kw
    Kpad = _round_up(Kc + 1, 8)
    B = 2 if N % 2 == 0 else 1

    w_mat = weight.transpose(0, 2, 3, 1).reshape(Cout, Kc)
    w_b = jnp.concatenate(
        [w_mat.astype(jnp.bfloat16),
         bias[:, None].astype(jnp.bfloat16),
         jnp.zeros((Cout, Kpad - Kc - 1), jnp.bfloat16)], axis=1)

    out = pl.pallas_call(
        functools.partial(_conv_kernel, offsets=offsets, cin=Cin, m=M,
                          kpad=Kpad, hw=H * W, ho=Ho, w=W, wo=Wo, bb=B),
        out_shape=jax.ShapeDtypeStruct((N, Cout, Ho * Wo), jnp.float32),
        grid=(N // B,),
        in_specs=[
            pl.BlockSpec((B, Cin, H * W), lambda n: (n, 0, 0)),
            pl.BlockSpec((Cout, Kpad), lambda n: (0, 0)),
        ],
        out_specs=pl.BlockSpec((B, Cout, Ho * Wo), lambda n: (n, 0, 0)),
        scratch_shapes=[
            pltpu.VMEM((B, Cin, H * W), jnp.bfloat16),
            pltpu.VMEM((Kpad, B * M), jnp.bfloat16),
        ],
        compiler_params=pltpu.CompilerParams(
            dimension_semantics=("parallel",),
        ),
    )(x.reshape(N, Cin, H * W), w_b)

    return out.reshape(N, Cout, Ho, Wo)
